# baseline (speedup 1.0000x reference)
import functools

import jax
import jax.numpy as jnp
from jax import lax
from jax.experimental import pallas as pl
from jax.experimental.pallas import tpu as pltpu
from jax.experimental.pallas import tpu_sc as plsc

B, HN, N1, DIM = 4096, 12, 128, 256
HP = 16
D2 = 2 * DIM
NP = 4
R = B * HN
NC, NS, L = 2, 16, 16
NW = NC * NS
CB = 4
CR = CB * HN
CRO = CB * HP
NSPLIT = 4
BH = B // NSPLIT

_mesh = plsc.VectorSubcoreMesh(core_axis_name="c", subcore_axis_name="s")
_sc_params = pltpu.CompilerParams(needs_layout_passes=False)


def _wid():
    return lax.axis_index("s") * NC + lax.axis_index("c")


def _make_scatter(b_lo, b_cnt):
    bpw = b_cnt // NW
    nch = bpw // CB

    @functools.partial(
        pl.kernel,
        out_type=jax.ShapeDtypeStruct((NP, b_cnt * HN, N1), jnp.float32),
        mesh=_mesh,
        scratch_types=[
            pltpu.VMEM((2 * CB, N1), jnp.int32),
            pltpu.VMEM((2 * CR, N1), jnp.float32),
            pltpu.VMEM((2 * CR, N1), jnp.float32),
            pltpu.VMEM((2 * NP, CR, N1), jnp.float32),
            pltpu.SemaphoreType.DMA((2,)),
            pltpu.SemaphoreType.DMA((2,)),
        ],
        compiler_params=_sc_params,
    )
    def scatter(rgb_hbm, tir_hbm, idx_hbm, vex_hbm,
                idx_v, rgb_v, tir_v, vex_v, si, so):
        base = b_lo + _wid() * bpw
        obase = _wid() * bpw
        zero16 = jnp.zeros((L,), jnp.float32)

        def in_copies(s, ci):
            b0 = base + ci * CB
            return (
                pltpu.make_async_copy(idx_hbm.at[pl.ds(b0, CB)],
                                      idx_v.at[pl.ds(s * CB, CB)], si.at[s]),
                pltpu.make_async_copy(rgb_hbm.at[pl.ds(b0 * HN, CR)],
                                      rgb_v.at[pl.ds(s * CR, CR)], si.at[s]),
                pltpu.make_async_copy(tir_hbm.at[pl.ds(b0 * HN, CR)],
                                      tir_v.at[pl.ds(s * CR, CR)], si.at[s]))

        def out_copies(s, ci):
            r0 = (obase + ci * CB) * HN
            return tuple(
                pltpu.make_async_copy(vex_v.at[s * NP + q],
                                      vex_hbm.at[q, pl.ds(r0, CR)], so.at[s])
                for q in range(NP))

        def compute(s):
            def zero_rows(r, carry):
                for q in range(NP):
                    for j in range(N1 // L):
                        vex_v[s * NP + q, r, pl.ds(j * L, L)] = zero16
                return carry

            lax.fori_loop(0, CR, zero_rows, 0)

            def one_b(bb, carry):
                sq = jnp.full((L,), s * NP, jnp.int32)
                rs = [jnp.full((L,), bb * HN + h, jnp.int32) for h in range(HN)]
                for g in range(N1 // L):
                    ig = idx_v[s * CB + bb, pl.ds(g * L, L)]
                    q0 = (ig >> 7) + sq
                    cc = ig & 127
                    q1 = q0 + 2
                    for h in range(HN):
                        plsc.store_scatter(
                            vex_v, [q0, rs[h], cc],
                            rgb_v[s * CR + bb * HN + h, pl.ds(g * L, L)])
                        plsc.store_scatter(
                            vex_v, [q1, rs[h], cc],
                            tir_v[s * CR + bb * HN + h, pl.ds(g * L, L)])
                return carry

            lax.fori_loop(0, CB, one_b, 0)

        for c in in_copies(0, 0):
            c.start()
        for c in in_copies(1, 1):
            c.start()

        def step(ci, carry):
            s = ci & 1
            for c in in_copies(s, ci):
                c.wait()

            @pl.when(ci >= 2)
            def _():
                for c in out_copies(s, ci):
                    c.wait()

            compute(s)
            for c in out_copies(s, ci):
                c.start()

            @pl.when(ci + 2 < nch)
            def _():
                for c in in_copies(s, ci + 2):
                    c.start()

            return carry

        lax.fori_loop(0, nch, step, 0)
        for c in out_copies(0, nch - 2):
            c.wait()
        for c in out_copies(1, nch - 1):
            c.wait()

    return scatter


def _make_gather(b_lo, b_cnt):
    bpw = b_cnt // NW
    nch = bpw // CB

    @functools.partial(
        pl.kernel,
        out_type=(
            jax.ShapeDtypeStruct((b_cnt * HP, N1), jnp.float32),
            jax.ShapeDtypeStruct((b_cnt * HP, N1), jnp.float32),
        ),
        mesh=_mesh,
        scratch_types=[
            pltpu.VMEM((2 * CB, N1), jnp.int32),
            pltpu.VMEM((2 * CRO, N1), jnp.float32),
            pltpu.VMEM((2 * CRO, N1), jnp.float32),
            pltpu.VMEM((2 * NP, CR, N1), jnp.float32),
            pltpu.SemaphoreType.DMA((2,)),
            pltpu.SemaphoreType.DMA((2,)),
        ],
        compiler_params=_sc_params,
    )
    def gather(y_hbm, idx_hbm, rgbo_hbm, tiro_hbm,
               idx_v, rgb_v, tir_v, y_v, si, so):
        base = b_lo + _wid() * bpw
        obase = _wid() * bpw

        def in_copies(s, ci):
            b0 = base + ci * CB
            r0 = (obase + ci * CB) * HN
            return (pltpu.make_async_copy(idx_hbm.at[pl.ds(b0, CB)],
                                          idx_v.at[pl.ds(s * CB, CB)], si.at[s]),
                    ) + tuple(
                pltpu.make_async_copy(y_hbm.at[q, pl.ds(r0, CR)],
                                      y_v.at[s * NP + q], si.at[s])
                for q in range(NP))

        def out_copies(s, ci):
            r0 = (obase + ci * CB) * HP
            return (
                pltpu.make_async_copy(rgb_v.at[pl.ds(s * CRO, CRO)],
                                      rgbo_hbm.at[pl.ds(r0, CRO)], so.at[s]),
                pltpu.make_async_copy(tir_v.at[pl.ds(s * CRO, CRO)],
                                      tiro_hbm.at[pl.ds(r0, CRO)], so.at[s]))

        def compute(s):
            def one_b(bb, carry):
                sq = jnp.full((L,), s * NP, jnp.int32)
                rs = [jnp.full((L,), bb * HN + h, jnp.int32) for h in range(HN)]
                for g in range(N1 // L):
                    ig = idx_v[s * CB + bb, pl.ds(g * L, L)]
                    q0 = (ig >> 7) + sq
                    cc = ig & 127
                    q1 = q0 + 2
                    for h in range(HN):
                        orow = s * CRO + bb * HP + h
                        rgb_v[orow, pl.ds(g * L, L)] = plsc.load_gather(
                            y_v, [q0, rs[h], cc])
                        tir_v[orow, pl.ds(g * L, L)] = plsc.load_gather(
                            y_v, [q1, rs[h], cc])
                return carry

            lax.fori_loop(0, CB, one_b, 0)

        for c in in_copies(0, 0):
            c.start()
        for c in in_copies(1, 1):
            c.start()

        def step(ci, carry):
            s = ci & 1
            for c in in_copies(s, ci):
                c.wait()

            @pl.when(ci >= 2)
            def _():
                for c in out_copies(s, ci):
                    c.wait()

            compute(s)
            for c in out_copies(s, ci):
                c.start()

            @pl.when(ci + 2 < nch)
            def _():
                for c in in_copies(s, ci + 2):
                    c.start()

            return carry

        lax.fori_loop(0, nch, step, 0)
        for c in out_copies(0, nch - 2):
            c.wait()
        for c in out_copies(1, nch - 1):
            c.wait()

    return gather


_RB = 1024


def _mlp_body(x_ref, lnw_ref, lnb_ref, w1_ref, b1_ref, w2_ref, b2_ref, y_ref):
    x = jnp.concatenate([x_ref[q] for q in range(NP)], axis=-1)
    mu = jnp.mean(x, axis=1, keepdims=True)
    xc = x - mu
    var = jnp.mean(xc * xc, axis=1, keepdims=True)
    xn = xc * lax.rsqrt(var + 1e-5) * lnw_ref[...] + lnb_ref[...]
    h = lax.dot_general(xn.astype(jnp.bfloat16),
                        w1_ref[...].astype(jnp.bfloat16),
                        (((1,), (1,)), ((), ())),
                        preferred_element_type=jnp.float32)
    h = jnp.maximum(h + b1_ref[...], 0.0)
    z = lax.dot_general(h.astype(jnp.bfloat16),
                        w2_ref[...].astype(jnp.bfloat16),
                        (((1,), (1,)), ((), ())),
                        preferred_element_type=jnp.float32)
    z = jax.nn.sigmoid(z + b2_ref[...])
    for q in range(NP):
        y_ref[q] = z[:, q * N1:(q + 1) * N1]


def _mlp(x, ln_w, ln_b, W1, b1, W2, b2):
    n = x.shape[1]
    grid = n // _RB
    return pl.pallas_call(
        _mlp_body,
        grid=(grid,),
        in_specs=[
            pl.BlockSpec((NP, _RB, N1), lambda i: (0, i, 0)),
            pl.BlockSpec((1, D2), lambda i: (0, 0)),
            pl.BlockSpec((1, D2), lambda i: (0, 0)),
            pl.BlockSpec((DIM, D2), lambda i: (0, 0)),
            pl.BlockSpec((1, DIM), lambda i: (0, 0)),
            pl.BlockSpec((D2, DIM), lambda i: (0, 0)),
            pl.BlockSpec((1, D2), lambda i: (0, 0)),
        ],
        out_specs=pl.BlockSpec((NP, _RB, N1), lambda i: (0, i, 0)),
        out_shape=jax.ShapeDtypeStruct((NP, n, N1), jnp.float32),
    )(x, ln_w.reshape(1, D2), ln_b.reshape(1, D2), W1, b1.reshape(1, DIM),
      W2, b2.reshape(1, D2))


_scatters = [_make_scatter(i * BH, BH) for i in range(NSPLIT)]
_gathers = [_make_gather(i * BH, BH) for i in range(NSPLIT)]


def kernel(attn_rgb_weight, attn_tir_weight, global_index_s, ln_w, ln_b,
           W1, b1, W2, b2):
    rgb = attn_rgb_weight.reshape(R, N1)
    tir = attn_tir_weight.reshape(R, N1)
    outs = []
    for i in range(NSPLIT):
        vex = _scatters[i](rgb, tir, global_index_s)
        y = _mlp(vex, ln_w, ln_b, W1, b1, W2, b2)
        outs.append(_gathers[i](y, global_index_s))
    rgb_o = jnp.concatenate([o[0] for o in outs], axis=0)
    tir_o = jnp.concatenate([o[1] for o in outs], axis=0)
    rgb_o = rgb_o.reshape(B, HP, N1)[:, :HN, :]
    tir_o = tir_o.reshape(B, HP, N1)[:, :HN, :]
    return rgb_o, tir_o


# speedup vs baseline: 1.0878x; 1.0878x over previous
import functools

import jax
import jax.numpy as jnp
from jax import lax
from jax.experimental import pallas as pl
from jax.experimental.pallas import tpu as pltpu
from jax.experimental.pallas import tpu_sc as plsc

B, HN, N1, DIM = 4096, 12, 128, 256
HP = 16
D2 = 2 * DIM
NP = 4
R = B * HN
NC, NS, L = 2, 16, 16
NW = NC * NS
CB = 4
CR = CB * HN
CRO = CB * HP
NSPLIT = 2
BH = B // NSPLIT

_mesh = plsc.VectorSubcoreMesh(core_axis_name="c", subcore_axis_name="s")
_sc_params = pltpu.CompilerParams(needs_layout_passes=False)


def _wid():
    return lax.axis_index("s") * NC + lax.axis_index("c")


def _make_scatter(b_lo, b_cnt):
    bpw = b_cnt // NW
    nch = bpw // CB

    @functools.partial(
        pl.kernel,
        out_type=jax.ShapeDtypeStruct((NP, b_cnt * HN, N1), jnp.float32),
        mesh=_mesh,
        scratch_types=[
            pltpu.VMEM((2 * CB, N1), jnp.int32),
            pltpu.VMEM((2 * CR, N1), jnp.float32),
            pltpu.VMEM((2 * CR, N1), jnp.float32),
            pltpu.VMEM((2 * NP, CR, N1), jnp.float32),
            pltpu.SemaphoreType.DMA((2,)),
            pltpu.SemaphoreType.DMA((2,)),
        ],
        compiler_params=_sc_params,
    )
    def scatter(rgb_hbm, tir_hbm, idx_hbm, vex_hbm,
                idx_v, rgb_v, tir_v, vex_v, si, so):
        base = b_lo + _wid() * bpw
        obase = _wid() * bpw
        zero16 = jnp.zeros((L,), jnp.float32)

        def in_copies(s, ci):
            b0 = base + ci * CB
            return (
                pltpu.make_async_copy(idx_hbm.at[pl.ds(b0, CB)],
                                      idx_v.at[pl.ds(s * CB, CB)], si.at[s]),
                pltpu.make_async_copy(rgb_hbm.at[pl.ds(b0 * HN, CR)],
                                      rgb_v.at[pl.ds(s * CR, CR)], si.at[s]),
                pltpu.make_async_copy(tir_hbm.at[pl.ds(b0 * HN, CR)],
                                      tir_v.at[pl.ds(s * CR, CR)], si.at[s]))

        def out_copies(s, ci):
            r0 = (obase + ci * CB) * HN
            return tuple(
                pltpu.make_async_copy(vex_v.at[s * NP + q],
                                      vex_hbm.at[q, pl.ds(r0, CR)], so.at[s])
                for q in range(NP))

        def compute(s):
            def zero_rows(r, carry):
                for q in range(NP):
                    for j in range(N1 // L):
                        vex_v[s * NP + q, r, pl.ds(j * L, L)] = zero16
                return carry

            lax.fori_loop(0, CR, zero_rows, 0)

            def one_b(bb, carry):
                sq = jnp.full((L,), s * NP, jnp.int32)
                rs = [jnp.full((L,), bb * HN + h, jnp.int32) for h in range(HN)]
                for g in range(N1 // L):
                    ig = idx_v[s * CB + bb, pl.ds(g * L, L)]
                    q0 = (ig >> 7) + sq
                    cc = ig & 127
                    q1 = q0 + 2
                    for h in range(HN):
                        plsc.store_scatter(
                            vex_v, [q0, rs[h], cc],
                            rgb_v[s * CR + bb * HN + h, pl.ds(g * L, L)])
                        plsc.store_scatter(
                            vex_v, [q1, rs[h], cc],
                            tir_v[s * CR + bb * HN + h, pl.ds(g * L, L)])
                return carry

            lax.fori_loop(0, CB, one_b, 0)

        for c in in_copies(0, 0):
            c.start()
        for c in in_copies(1, 1):
            c.start()

        def step(ci, carry):
            s = ci & 1
            for c in in_copies(s, ci):
                c.wait()

            @pl.when(ci >= 2)
            def _():
                for c in out_copies(s, ci):
                    c.wait()

            compute(s)
            for c in out_copies(s, ci):
                c.start()

            @pl.when(ci + 2 < nch)
            def _():
                for c in in_copies(s, ci + 2):
                    c.start()

            return carry

        lax.fori_loop(0, nch, step, 0)
        for c in out_copies(0, nch - 2):
            c.wait()
        for c in out_copies(1, nch - 1):
            c.wait()

    return scatter


def _make_gather(b_lo, b_cnt):
    bpw = b_cnt // NW
    nch = bpw // CB

    @functools.partial(
        pl.kernel,
        out_type=(
            jax.ShapeDtypeStruct((b_cnt * HP, N1), jnp.float32),
            jax.ShapeDtypeStruct((b_cnt * HP, N1), jnp.float32),
        ),
        mesh=_mesh,
        scratch_types=[
            pltpu.VMEM((2 * CB, N1), jnp.int32),
            pltpu.VMEM((2 * CRO, N1), jnp.float32),
            pltpu.VMEM((2 * CRO, N1), jnp.float32),
            pltpu.VMEM((2 * NP, CR, N1), jnp.float32),
            pltpu.SemaphoreType.DMA((2,)),
            pltpu.SemaphoreType.DMA((2,)),
        ],
        compiler_params=_sc_params,
    )
    def gather(y_hbm, idx_hbm, rgbo_hbm, tiro_hbm,
               idx_v, rgb_v, tir_v, y_v, si, so):
        base = b_lo + _wid() * bpw
        obase = _wid() * bpw

        def in_copies(s, ci):
            b0 = base + ci * CB
            r0 = (obase + ci * CB) * HN
            return (pltpu.make_async_copy(idx_hbm.at[pl.ds(b0, CB)],
                                          idx_v.at[pl.ds(s * CB, CB)], si.at[s]),
                    ) + tuple(
                pltpu.make_async_copy(y_hbm.at[q, pl.ds(r0, CR)],
                                      y_v.at[s * NP + q], si.at[s])
                for q in range(NP))

        def out_copies(s, ci):
            r0 = (obase + ci * CB) * HP
            return (
                pltpu.make_async_copy(rgb_v.at[pl.ds(s * CRO, CRO)],
                                      rgbo_hbm.at[pl.ds(r0, CRO)], so.at[s]),
                pltpu.make_async_copy(tir_v.at[pl.ds(s * CRO, CRO)],
                                      tiro_hbm.at[pl.ds(r0, CRO)], so.at[s]))

        def compute(s):
            def one_b(bb, carry):
                sq = jnp.full((L,), s * NP, jnp.int32)
                rs = [jnp.full((L,), bb * HN + h, jnp.int32) for h in range(HN)]
                for g in range(N1 // L):
                    ig = idx_v[s * CB + bb, pl.ds(g * L, L)]
                    q0 = (ig >> 7) + sq
                    cc = ig & 127
                    q1 = q0 + 2
                    for h in range(HN):
                        orow = s * CRO + bb * HP + h
                        rgb_v[orow, pl.ds(g * L, L)] = plsc.load_gather(
                            y_v, [q0, rs[h], cc])
                        tir_v[orow, pl.ds(g * L, L)] = plsc.load_gather(
                            y_v, [q1, rs[h], cc])
                return carry

            lax.fori_loop(0, CB, one_b, 0)

        for c in in_copies(0, 0):
            c.start()
        for c in in_copies(1, 1):
            c.start()

        def step(ci, carry):
            s = ci & 1
            for c in in_copies(s, ci):
                c.wait()

            @pl.when(ci >= 2)
            def _():
                for c in out_copies(s, ci):
                    c.wait()

            compute(s)
            for c in out_copies(s, ci):
                c.start()

            @pl.when(ci + 2 < nch)
            def _():
                for c in in_copies(s, ci + 2):
                    c.start()

            return carry

        lax.fori_loop(0, nch, step, 0)
        for c in out_copies(0, nch - 2):
            c.wait()
        for c in out_copies(1, nch - 1):
            c.wait()

    return gather


_RB = 1024


def _mlp_body(x_ref, lnw_ref, lnb_ref, w1_ref, b1_ref, w2_ref, b2_ref, y_ref):
    x = jnp.concatenate([x_ref[q] for q in range(NP)], axis=-1)
    mu = jnp.mean(x, axis=1, keepdims=True)
    xc = x - mu
    var = jnp.mean(xc * xc, axis=1, keepdims=True)
    xn = xc * lax.rsqrt(var + 1e-5) * lnw_ref[...] + lnb_ref[...]
    h = lax.dot_general(xn.astype(jnp.bfloat16),
                        w1_ref[...].astype(jnp.bfloat16),
                        (((1,), (1,)), ((), ())),
                        preferred_element_type=jnp.float32)
    h = jnp.maximum(h + b1_ref[...], 0.0)
    z = lax.dot_general(h.astype(jnp.bfloat16),
                        w2_ref[...].astype(jnp.bfloat16),
                        (((1,), (1,)), ((), ())),
                        preferred_element_type=jnp.float32)
    z = jax.nn.sigmoid(z + b2_ref[...])
    for q in range(NP):
        y_ref[q] = z[:, q * N1:(q + 1) * N1]


def _mlp(x, ln_w, ln_b, W1, b1, W2, b2):
    n = x.shape[1]
    grid = n // _RB
    return pl.pallas_call(
        _mlp_body,
        grid=(grid,),
        in_specs=[
            pl.BlockSpec((NP, _RB, N1), lambda i: (0, i, 0)),
            pl.BlockSpec((1, D2), lambda i: (0, 0)),
            pl.BlockSpec((1, D2), lambda i: (0, 0)),
            pl.BlockSpec((DIM, D2), lambda i: (0, 0)),
            pl.BlockSpec((1, DIM), lambda i: (0, 0)),
            pl.BlockSpec((D2, DIM), lambda i: (0, 0)),
            pl.BlockSpec((1, D2), lambda i: (0, 0)),
        ],
        out_specs=pl.BlockSpec((NP, _RB, N1), lambda i: (0, i, 0)),
        out_shape=jax.ShapeDtypeStruct((NP, n, N1), jnp.float32),
    )(x, ln_w.reshape(1, D2), ln_b.reshape(1, D2), W1, b1.reshape(1, DIM),
      W2, b2.reshape(1, D2))


_scatters = [_make_scatter(i * BH, BH) for i in range(NSPLIT)]
_gathers = [_make_gather(i * BH, BH) for i in range(NSPLIT)]


def kernel(attn_rgb_weight, attn_tir_weight, global_index_s, ln_w, ln_b,
           W1, b1, W2, b2):
    rgb = attn_rgb_weight.reshape(R, N1)
    tir = attn_tir_weight.reshape(R, N1)
    outs = []
    for i in range(NSPLIT):
        vex = _scatters[i](rgb, tir, global_index_s)
        y = _mlp(vex, ln_w, ln_b, W1, b1, W2, b2)
        outs.append(_gathers[i](y, global_index_s))
    rgb_o = jnp.concatenate([o[0] for o in outs], axis=0)
    tir_o = jnp.concatenate([o[1] for o in outs], axis=0)
    rgb_o = rgb_o.reshape(B, HP, N1)[:, :HN, :]
    tir_o = tir_o.reshape(B, HP, N1)[:, :HN, :]
    return rgb_o, tir_o
